# baseline (speedup 1.0000x reference)
import functools

import jax
import jax.numpy as jnp
from jax import lax
from jax.experimental import pallas as pl
from jax.experimental.pallas import tpu as pltpu
from jax.experimental.pallas import tpu_sc as plsc

DIM = 64
L = 16
NC, NS = 2, 16
NW = NC * NS
BATCH_ROWS = 16

_PARAMS = pltpu.CompilerParams(needs_layout_passes=False,
                               use_tc_tiling_on_sc=True)
_MESH = plsc.VectorSubcoreMesh(core_axis_name="c", subcore_axis_name="s")


def _rsqrt(x):
    i = lax.bitcast_convert_type(x, jnp.int32)
    i = jnp.int32(0x5F3759DF) - lax.shift_right_logical(i, 1)
    y = lax.bitcast_convert_type(i, jnp.float32)
    for _ in range(4):
        y = y * (1.5 - 0.5 * x * y * y)
    return y


def _inv_norm(ss):
    rs = _rsqrt(jnp.maximum(ss, 1e-30))
    n = ss * rs
    return 1.0 / jnp.maximum(n, 1e-12)


def _sqrt(x):
    xc = jnp.maximum(x, 0.0)
    return xc * _rsqrt(jnp.maximum(xc, 1e-30))


def _iota():
    return lax.iota(jnp.int32, L)


def _make_extract(n_ent, n_idx):
    nfull = n_ent // 128
    ntail = n_ent % 128
    n_rows = n_idx + 2 * BATCH_ROWS
    trash = n_idx
    q, r = divmod(nfull, NW)
    quarter = n_idx // 4
    HMAX = n_idx // 2
    SB = 4

    @functools.partial(
        pl.kernel,
        mesh=_MESH,
        compiler_params=_PARAMS,
        out_type=jax.ShapeDtypeStruct((n_rows, 2 * DIM), jnp.float32),
        scratch_types=(
            [pltpu.VMEM((quarter,), jnp.int32),
             pltpu.VMEM((HMAX,), jnp.int32),
             pltpu.VMEM((272,), jnp.int32),
             pltpu.VMEM((272,), jnp.int32),
             pltpu.VMEM((272,), jnp.int32)]
            + [pltpu.VMEM((DIM, SB * 2 * DIM), jnp.float32) for _ in range(2)]
            + [pltpu.VMEM((BATCH_ROWS, 2 * DIM), jnp.float32) for _ in range(2)]
            + [pltpu.VMEM((BATCH_ROWS,), jnp.int32) for _ in range(2)]
            + [pltpu.SemaphoreType.DMA for _ in range(4)]
        ),
    )
    def extract_kernel(ent_t, tailp, gidx_hbm, rows_out,
                       ichunk, hits, hist, offs, cur, *rest):
        superb = rest[0:2]
        stage = rest[2:4]
        slotb = rest[4:6]
        semb = rest[6:8]
        semst = rest[8:10]
        wid = lax.axis_index("s") * NC + lax.axis_index("c")
        blk0 = wid * q + jnp.minimum(wid, r)
        nblk = q + jnp.where(wid < r, 1, 0)
        nbuckets = nblk + (jnp.where(jnp.equal(wid, NW - 1), 1, 0)
                           if ntail else 0)
        nsuper = (nbuckets + SB - 1) // SB
        iota = _iota()
        ones = jnp.full((L,), 1, jnp.int32)

        for k in range(16):
            hist[pl.ds(k * L, L)] = jnp.zeros((L,), jnp.int32)

        def classify(v):
            bofs = lax.shift_right_logical(v, 7) - blk0
            m = (bofs >= 0) & (bofs < nbuckets)
            bc = jnp.clip(bofs, 0, 255)
            return m, bc

        for h in range(4):
            pltpu.sync_copy(gidx_hbm.at[pl.ds(h * quarter, quarter)], ichunk)

            def hist_body(k, cc):
                v = ichunk[pl.ds(k * L, L)]
                m, bc = classify(v)
                plsc.addupdate_scatter(hist, [bc], ones, mask=m)
                return cc
            lax.fori_loop(0, quarter // L, hist_body, 0)

        carry = jnp.int32(0)
        for k in range(16):
            hv = hist[pl.ds(k * L, L)]
            incl = plsc.cumsum(hv) + carry
            excl = incl - hv
            offs[pl.ds(k * L, L)] = excl
            cur[pl.ds(k * L, L)] = excl
            carry = incl[L - 1]

        for h in range(4):
            pltpu.sync_copy(gidx_hbm.at[pl.ds(h * quarter, quarter)], ichunk)

            def place_body(k, cc):
                v = ichunk[pl.ds(k * L, L)]
                m, bc = classify(v)
                slotv = jnp.full((L,), h * quarter, jnp.int32) + k * L + iota
                rec = slotv | ((v & 127) << 16)
                cnt, lastm = plsc.scan_count(bc, mask=m)
                base = plsc.load_gather(cur, [bc])
                pos = jnp.clip(base + cnt - 1, 0, HMAX - 1)
                plsc.store_scatter(hits, [pos], rec, mask=m)
                plsc.store_scatter(cur, [bc], base + cnt, mask=m & lastm)
                return cc
            lax.fori_loop(0, quarter // L, place_body, 0)

        def fire_super(p, so):
            colb = (blk0 + so * SB) * 128

            @pl.when(colb + SB * 128 <= n_ent)
            def _():
                for i in range(8):
                    pltpu.async_copy(
                        ent_t.at[pl.ds(8 * i, 8), pl.ds(colb, SB * 128)],
                        superb[p].at[pl.ds(8 * i, 8), :], semb[p])

            @pl.when(colb + SB * 128 > n_ent)
            def _():
                pltpu.async_copy(tailp, superb[p].at[:, pl.ds(0, 128)],
                                 semb[p])

        def drain_super(p, so):
            colb = (blk0 + so * SB) * 128

            @pl.when(colb + SB * 128 <= n_ent)
            def _():
                for i in range(8):
                    pltpu.make_async_copy(
                        ent_t.at[pl.ds(8 * i, 8), pl.ds(colb, SB * 128)],
                        superb[p].at[pl.ds(8 * i, 8), :], semb[p]).wait()

            @pl.when(colb + SB * 128 > n_ent)
            def _():
                pltpu.make_async_copy(tailp, superb[p].at[:, pl.ds(0, 128)],
                                      semb[p]).wait()

        def fire_st(p):
            pltpu.async_copy(stage[p], rows_out.at[slotb[p]], semst[p])

        def drain_st(p):
            pltpu.make_async_copy(stage[p], rows_out.at[slotb[p]],
                                  semst[p]).wait()

        for p in range(2):
            fire_super(p, p)

        def process(b, p, s4, fl):
            start = offs[pl.ds(b, L)][0]
            end = cur[pl.ds(b, L)][0]
            nbt = lax.shift_right_logical(end - start + (BATCH_ROWS - 1), 4)
            npr = lax.shift_right_logical(nbt + 1, 1)

            def hit_pair(ho, fl):
                for s2 in range(2):
                    ib = ho * 2 + s2

                    @pl.when(fl[s2] > 0)
                    def _():
                        drain_st(s2)

                    hpos = start + ib * BATCH_ROWS + iota
                    validm = hpos < end
                    recv = plsc.load_gather(
                        hits, [jnp.clip(hpos, 0, HMAX - 1)])
                    slotv = jnp.where(validm, recv & 0xFFFF, trash)
                    cvec = (lax.shift_right_logical(recv, 16) & 127) + s4 * 128
                    slotb[s2][...] = slotv
                    for d in range(DIM):
                        val = plsc.load_gather(
                            superb[p], [jnp.full((L,), d, jnp.int32), cvec])
                        plsc.store_scatter(
                            stage[s2], [iota, jnp.full((L,), d, jnp.int32)],
                            val)
                    fire_st(s2)
                return (jnp.int32(1), jnp.int32(1))

            return lax.fori_loop(0, npr, hit_pair, fl)

        def run_super(so, p, fl):
            drain_super(p, so)
            for s4 in range(SB):
                b = so * SB + s4
                fl = lax.cond(b < nbuckets,
                              lambda fl, b=b, s4=s4: process(b, p, s4, fl),
                              lambda fl: fl,
                              fl)

            @pl.when(so + 2 < nsuper)
            def _():
                fire_super(p, so + 2)
            return fl

        def super_pair(io, fl):
            for sp in range(2):
                so = io * 2 + sp
                fl = lax.cond(so < nsuper,
                              lambda fl, so=so, sp=sp: run_super(so, sp, fl),
                              lambda fl: fl,
                              fl)
            return fl

        fl = lax.fori_loop(0, (nsuper + 1) // 2, super_pair,
                           (jnp.int32(0), jnp.int32(0)))

        @pl.when(fl[0] > 0)
        def _():
            drain_st(0)

        @pl.when(fl[1] > 0)
        def _():
            drain_st(1)

    return extract_kernel


def _make_energy(batch, chunk):
    ept = batch // NW
    nchunk = ept // chunk
    groups = chunk // L
    n_rows = 4 * batch + 2 * BATCH_ROWS

    @functools.partial(
        pl.kernel,
        mesh=_MESH,
        compiler_params=_PARAMS,
        out_type=jax.ShapeDtypeStruct((NW * L,), jnp.float32),
        scratch_types=(
            [pltpu.VMEM((ept,), jnp.int32),
             pltpu.VMEM((ept,), jnp.int32)]
            + [pltpu.VMEM((chunk,), jnp.int32) for _ in range(2)]
            + [pltpu.VMEM((chunk, 2 * DIM), jnp.float32) for _ in range(10)]
            + [pltpu.VMEM((L,), jnp.float32),
               pltpu.SemaphoreType.DMA,
               pltpu.SemaphoreType.DMA]
        ),
    )
    def energy_kernel(rows_hbm, rel_hbm, ridx_hbm, rcolb_hbm, out_hbm,
                      ridx_v, rcolb_v, *rest):
        idxc = rest[0:2]
        rows_v = [rest[2 + b * 5:2 + (b + 1) * 5] for b in range(2)]
        acc_v, sem0, sem1 = rest[12:]
        wid = lax.axis_index("s") * NC + lax.axis_index("c")
        base0 = wid * ept
        sems = [sem0, sem1]
        iota = _iota()

        pltpu.sync_copy(ridx_hbm.at[pl.ds(base0, ept)], ridx_v)
        pltpu.sync_copy(rcolb_hbm.at[pl.ds(base0, ept)], rcolb_v)

        def copies(b, c):
            return ([(rows_hbm.at[pl.ds(j * batch + base0 + c * chunk, chunk)],
                      rows_v[b][j]) for j in range(4)]
                    + [(rel_hbm.at[idxc[b]], rows_v[b][4])])

        def fire(b, c):
            for k in range(chunk // L):
                idxc[b][pl.ds(k * L, L)] = ridx_v[pl.ds(c * chunk + k * L, L)]
            for src, dst in copies(b, c):
                pltpu.async_copy(src, dst, sems[b])

        def drain(b, c):
            for src, dst in copies(b, c):
                pltpu.make_async_copy(src, dst, sems[b]).wait()

        def make_group_body(b, c):
            def group_body(g, acc):
                row0 = jnp.full((L,), g * L, jnp.int32) + iota
                cbr = plsc.load_gather(rcolb_v, [c * chunk + row0])
                z = jnp.zeros((L,), jnp.float32)
                ss_hp = ss_tp = ss_hn = ss_tn = ss_r = z
                d_hp_r = d_hp_tp = d_r_tp = d_hn_r = d_hn_tn = d_r_tn = z
                for d in range(DIM):
                    dv = jnp.full((L,), d, jnp.int32)
                    hp = plsc.load_gather(rows_v[b][0], [row0, dv])
                    tp = plsc.load_gather(rows_v[b][1], [row0, dv])
                    hn = plsc.load_gather(rows_v[b][2], [row0, dv])
                    tn = plsc.load_gather(rows_v[b][3], [row0, dv])
                    r = plsc.load_gather(rows_v[b][4], [row0, cbr + dv])
                    ss_hp += hp * hp
                    ss_tp += tp * tp
                    ss_hn += hn * hn
                    ss_tn += tn * tn
                    ss_r += r * r
                    d_hp_r += hp * r
                    d_hp_tp += hp * tp
                    d_r_tp += r * tp
                    d_hn_r += hn * r
                    d_hn_tn += hn * tn
                    d_r_tn += r * tn
                ihp, itp = _inv_norm(ss_hp), _inv_norm(ss_tp)
                ihn, itn = _inv_norm(ss_hn), _inv_norm(ss_tn)
                ir = _inv_norm(ss_r)
                rr = ss_r * ir * ir
                e2p = (ss_hp * ihp * ihp + rr + ss_tp * itp * itp
                       + 2.0 * (d_hp_r * ihp * ir - d_hp_tp * ihp * itp
                                - d_r_tp * ir * itp))
                e2n = (ss_hn * ihn * ihn + rr + ss_tn * itn * itn
                       + 2.0 * (d_hn_r * ihn * ir - d_hn_tn * ihn * itn
                                - d_r_tn * ir * itn))
                loss = jnp.maximum(1.0 + _sqrt(e2p) - _sqrt(e2n), 0.0)
                return acc + loss
            return group_body

        acc = jnp.zeros((L,), jnp.float32)
        for c in range(2):
            fire(c, c)

        def chunk_pair(i, acc):
            for b in range(2):
                c = i * 2 + b
                drain(b, c)
                acc = lax.fori_loop(0, groups, make_group_body(b, c), acc)

                @pl.when(c + 2 < nchunk)
                def _():
                    fire(b, c + 2)
            return acc

        acc = lax.fori_loop(0, nchunk // 2, chunk_pair, acc)

        acc_v[...] = acc
        pltpu.sync_copy(acc_v, out_hbm.at[pl.ds(wid * L, L)])

    return energy_kernel


def kernel(ent_emb, rel_emb, pos_pairs, neg_pairs, rels):
    batch = pos_pairs.shape[0]
    n_ent, dim = ent_emb.shape
    nfull = n_ent // 128
    ent_t = ent_emb.T
    tailp = jnp.pad(ent_emb[nfull * 128:].T,
                    ((0, 0), (0, 128 - n_ent % 128)))
    gidx = jnp.stack([pos_pairs[:, 0], pos_pairs[:, 1],
                      neg_pairs[:, 0], neg_pairs[:, 1]],
                     axis=0).astype(jnp.int32).reshape(-1)
    rows = _make_extract(n_ent, 4 * batch)(ent_t, tailp, gidx)
    rel2 = rel_emb.reshape(rel_emb.shape[0] // 2, 2 * DIM)
    rr = rels[:, 0].astype(jnp.int32)
    partial = _make_energy(batch, 64)(rows, rel2, rr >> 1, (rr & 1) << 6)
    return jnp.sum(partial) / batch


# speedup vs baseline: 10.6197x; 10.6197x over previous
import functools

import jax
import jax.numpy as jnp
from jax import lax
from jax.experimental import pallas as pl
from jax.experimental.pallas import tpu as pltpu
from jax.experimental.pallas import tpu_sc as plsc

DIM = 64
L = 16
NC, NS = 2, 16
NW = NC * NS
NBUF = 2


def _rsqrt(x):
    i = lax.bitcast_convert_type(x, jnp.int32)
    i = jnp.int32(0x5F3759DF) - lax.shift_right_logical(i, 1)
    y = lax.bitcast_convert_type(i, jnp.float32)
    for _ in range(4):
        y = y * (1.5 - 0.5 * x * y * y)
    return y


def _inv_norm(ss):
    rs = _rsqrt(jnp.maximum(ss, 1e-30))
    n = ss * rs
    return 1.0 / jnp.maximum(n, 1e-12)


def _sqrt(x):
    xc = jnp.maximum(x, 0.0)
    return xc * _rsqrt(jnp.maximum(xc, 1e-30))


def _make_kernel(batch, chunk):
    ept = batch // NW
    nchunk = ept // chunk
    groups = chunk // L
    mesh = plsc.VectorSubcoreMesh(core_axis_name="c", subcore_axis_name="s")

    @functools.partial(
        pl.kernel,
        mesh=mesh,
        compiler_params=pltpu.CompilerParams(
            needs_layout_passes=False, use_tc_tiling_on_sc=True),
        out_type=jax.ShapeDtypeStruct((NW * L,), jnp.float32),
        scratch_types=(
            [pltpu.VMEM((5 * ept,), jnp.int32),
             pltpu.VMEM((5 * ept,), jnp.int32)]
            + [pltpu.VMEM((chunk,), jnp.int32) for _ in range(NBUF * 5)]
            + [pltpu.VMEM((chunk, 2 * DIM), jnp.float32) for _ in range(NBUF * 5)]
            + [pltpu.VMEM((L,), jnp.float32),
               pltpu.SemaphoreType.DMA,
               pltpu.SemaphoreType.DMA]
        ),
    )
    def transe_kernel(ent_hbm, rel_hbm, gidx_hbm, colb_hbm, out_hbm,
                      idx_v, colb_v, *rest):
        idxc = [rest[b * 5:(b + 1) * 5] for b in range(NBUF)]
        rows_v = [rest[NBUF * 5 + b * 5:NBUF * 5 + (b + 1) * 5] for b in range(NBUF)]
        acc_v, sem0, sem1 = rest[2 * NBUF * 5:]
        wid = lax.axis_index("s") * NC + lax.axis_index("c")
        base0 = wid * ept
        sems = [sem0, sem1]

        for j in range(5):
            pltpu.sync_copy(gidx_hbm.at[pl.ds(j * batch + base0, ept)],
                            idx_v.at[pl.ds(j * ept, ept)])
            pltpu.sync_copy(colb_hbm.at[pl.ds(j * batch + base0, ept)],
                            colb_v.at[pl.ds(j * ept, ept)])

        def copies(b):
            return ([(ent_hbm.at[idxc[b][j]], rows_v[b][j]) for j in range(4)]
                    + [(rel_hbm.at[idxc[b][4]], rows_v[b][4])])

        def fire(b, c):
            for j in range(5):
                for k in range(chunk // L):
                    idxc[b][j][pl.ds(k * L, L)] = (
                        idx_v[pl.ds(j * ept + c * chunk + k * L, L)])
            for src, dst in copies(b):
                pltpu.async_copy(src, dst, sems[b])

        def drain(b):
            for src, dst in copies(b):
                pltpu.make_async_copy(src, dst, sems[b]).wait()

        def make_group_body(b, c):
            def group_body(g, acc):
                row0 = jnp.full((L,), g * L, jnp.int32) + lax.iota(jnp.int32, L)
                gbase = c * chunk + row0
                cb = [plsc.load_gather(colb_v, [jnp.full((L,), j * ept, jnp.int32) + gbase])
                      for j in range(5)]
                z = jnp.zeros((L,), jnp.float32)
                ss_hp = ss_tp = ss_hn = ss_tn = ss_r = z
                d_hp_r = d_hp_tp = d_r_tp = d_hn_r = d_hn_tn = d_r_tn = z
                for d in range(DIM):
                    dv = jnp.full((L,), d, jnp.int32)
                    hp = plsc.load_gather(rows_v[b][0], [row0, cb[0] + dv])
                    tp = plsc.load_gather(rows_v[b][1], [row0, cb[1] + dv])
                    hn = plsc.load_gather(rows_v[b][2], [row0, cb[2] + dv])
                    tn = plsc.load_gather(rows_v[b][3], [row0, cb[3] + dv])
                    r = plsc.load_gather(rows_v[b][4], [row0, cb[4] + dv])
                    ss_hp += hp * hp
                    ss_tp += tp * tp
                    ss_hn += hn * hn
                    ss_tn += tn * tn
                    ss_r += r * r
                    d_hp_r += hp * r
                    d_hp_tp += hp * tp
                    d_r_tp += r * tp
                    d_hn_r += hn * r
                    d_hn_tn += hn * tn
                    d_r_tn += r * tn
                ihp, itp = _inv_norm(ss_hp), _inv_norm(ss_tp)
                ihn, itn = _inv_norm(ss_hn), _inv_norm(ss_tn)
                ir = _inv_norm(ss_r)
                rr = ss_r * ir * ir
                e2p = (ss_hp * ihp * ihp + rr + ss_tp * itp * itp
                       + 2.0 * (d_hp_r * ihp * ir - d_hp_tp * ihp * itp - d_r_tp * ir * itp))
                e2n = (ss_hn * ihn * ihn + rr + ss_tn * itn * itn
                       + 2.0 * (d_hn_r * ihn * ir - d_hn_tn * ihn * itn - d_r_tn * ir * itn))
                loss = jnp.maximum(1.0 + _sqrt(e2p) - _sqrt(e2n), 0.0)
                return acc + loss
            return group_body

        acc = jnp.zeros((L,), jnp.float32)
        for c in range(min(NBUF, nchunk)):
            fire(c, c)

        def chunk_pair(i, acc):
            for b in range(NBUF):
                c = i * NBUF + b
                drain(b)
                acc = lax.fori_loop(0, groups, make_group_body(b, c), acc)

                @pl.when(c + NBUF < nchunk)
                def _():
                    fire(b, c + NBUF)
            return acc

        acc = lax.fori_loop(0, nchunk // NBUF, chunk_pair, acc)

        acc_v[...] = acc
        pltpu.sync_copy(acc_v, out_hbm.at[pl.ds(wid * L, L)])

    return transe_kernel


def kernel(ent_emb, rel_emb, pos_pairs, neg_pairs, rels):
    batch = pos_pairs.shape[0]
    n_ent = ent_emb.shape[0]
    n_rel = rel_emb.shape[0]
    ent2 = ent_emb.reshape(n_ent // 2, 2 * DIM)
    rel2 = rel_emb.reshape(n_rel // 2, 2 * DIM)
    idx = jnp.stack([pos_pairs[:, 0], pos_pairs[:, 1],
                     neg_pairs[:, 0], neg_pairs[:, 1],
                     rels[:, 0]], axis=0).astype(jnp.int32)
    gidx = (idx >> 1).reshape(-1)
    colb = ((idx & 1) << 6).reshape(-1)
    partial = _make_kernel(batch, 64)(ent2, rel2, gidx, colb)
    return jnp.sum(partial) / batch
